# baseline (speedup 1.0000x reference)
import functools

import jax
import jax.numpy as jnp
from jax import lax
from jax.experimental import pallas as pl
from jax.experimental.pallas import tpu as pltpu
from jax.experimental.pallas import tpu_sc as plsc

N = 10000
E = 320000
D = 128
NLAYER = 3

NC = 2
NS = 16
NW = NC * NS

K = 128
GI = 8
NGI = 10
CH = GI * NGI
EPT = CH * K
E_PAD = EPT * NW

NP = 10112
RPT = NP // NS
BLK = 128


def _sc_agg_body(h_hbm, ei_hbm, zeros_hbm, out_hbm, ib, rows, acc,
                 isem0, isem1, gsem0, gsem1):
    isems = (isem0, isem1)
    gsems = (gsem0, gsem1)
    cid = lax.axis_index("c")
    sid = lax.axis_index("s")
    tid = sid * NC + cid

    pltpu.sync_copy(ei_hbm.at[tid, pl.ds(0, GI)], ib.at[0])
    pltpu.sync_copy(zeros_hbm.at[pl.ds(sid * RPT, RPT)],
                    acc.at[pl.ds(sid * RPT, RPT)])
    plsc.subcore_barrier()

    for b in range(2):
        pltpu.async_copy(h_hbm.at[ib.at[0, b, 0]], rows.at[b], gsems[b])

    def outer(go, carry):
        for half in range(2):
            g = go * 2 + half
            bi, bo = half, 1 - half

            @pl.when(g < NGI - 1)
            def _():
                pltpu.async_copy(ei_hbm.at[tid, pl.ds((g + 1) * GI, GI)],
                                 ib.at[bo], isems[bo])

            for j in range(GI):
                b = j % 2
                pltpu.make_async_copy(h_hbm.at[ib.at[bi, j, 0]],
                                      rows.at[b], gsems[b]).wait()
                pltpu.sync_copy(rows.at[b], acc.at[ib.at[bi, j, 1]],
                                add=True)
                if j == GI - 2:
                    @pl.when(g < NGI - 1)
                    def _():
                        pltpu.make_async_copy(
                            ei_hbm.at[tid, pl.ds((g + 1) * GI, GI)],
                            ib.at[bo], isems[bo]).wait()
                if j < GI - 2:
                    pltpu.async_copy(h_hbm.at[ib.at[bi, j + 2, 0]],
                                     rows.at[b], gsems[b])
                else:
                    @pl.when(g < NGI - 1)
                    def _():
                        pltpu.async_copy(
                            h_hbm.at[ib.at[bo, j + 2 - GI, 0]],
                            rows.at[b], gsems[b])
        return carry

    lax.fori_loop(0, NGI // 2, outer, 0)
    plsc.subcore_barrier()

    pltpu.sync_copy(acc.at[pl.ds(sid * RPT, RPT)],
                    out_hbm.at[cid, pl.ds(sid * RPT, RPT)])


@functools.cache
def _sc_agg():
    mesh = plsc.VectorSubcoreMesh(
        core_axis_name="c", subcore_axis_name="s",
        num_cores=NC, num_subcores=NS)
    return pl.kernel(
        _sc_agg_body,
        out_type=jax.ShapeDtypeStruct((NC, NP, D), jnp.float32),
        mesh=mesh,
        scratch_types=[
            pltpu.VMEM((2, GI, 2, K), jnp.int32),
            pltpu.VMEM((2, K, D), jnp.float32),
            pltpu.VMEM_SHARED((NP, D), jnp.float32),
            pltpu.SemaphoreType.DMA,
            pltpu.SemaphoreType.DMA,
            pltpu.SemaphoreType.DMA,
            pltpu.SemaphoreType.DMA,
        ],
    )


def _ln(x, g, b):
    mu = jnp.mean(x, axis=-1, keepdims=True)
    var = jnp.mean((x - mu) * (x - mu), axis=-1, keepdims=True)
    return (x - mu) * lax.rsqrt(var + 1e-5) * g + b


def _mlp_body_inner(h_ref, a0_ref, a1_ref, w1_ref, b1_ref, w2_ref, b2_ref,
                    ng_ref, nb_ref, sg_ref, sb_ref, eps_ref, out_ref):
    h = h_ref[...]
    z = (1.0 + eps_ref[0]) * h + a0_ref[...] + a1_ref[...]
    z = jnp.maximum(
        jnp.dot(z, w1_ref[...], preferred_element_type=jnp.float32)
        + b1_ref[...], 0.0)
    z = jnp.dot(z, w2_ref[...], preferred_element_type=jnp.float32) + b2_ref[...]
    z = _ln(z, ng_ref[...], nb_ref[...])
    z = jnp.maximum(_ln(z, sg_ref[...], sb_ref[...]), 0.0)
    out_ref[...] = z + h


def _mlp_body_last(h_ref, a0_ref, a1_ref, w1_ref, b1_ref, w2_ref, b2_ref,
                   ng_ref, nb_ref, eps_ref, out_ref):
    h = h_ref[...]
    z = (1.0 + eps_ref[0]) * h + a0_ref[...] + a1_ref[...]
    z = jnp.maximum(
        jnp.dot(z, w1_ref[...], preferred_element_type=jnp.float32)
        + b1_ref[...], 0.0)
    z = jnp.dot(z, w2_ref[...], preferred_element_type=jnp.float32) + b2_ref[...]
    z = _ln(z, ng_ref[...], nb_ref[...])
    out_ref[...] = z + h


def _row_spec():
    return pl.BlockSpec((BLK, D), lambda i: (i, 0))


def _full_spec(shape):
    nd = len(shape)
    return pl.BlockSpec(shape, lambda i: (0,) * nd)


def _tc_mlp(inner, h, a0, a1, w1, b1, w2, b2, ng, nb, sg, sb, eps):
    vecs = [v.reshape(1, D) for v in (b1, b2, ng, nb)]
    body = _mlp_body_last
    if inner:
        vecs += [sg.reshape(1, D), sb.reshape(1, D)]
        body = _mlp_body_inner
    in_specs = (
        [_row_spec(), _row_spec(), _row_spec(),
         _full_spec((D, D)), _full_spec((1, D)),
         _full_spec((D, D)), _full_spec((1, D)),
         _full_spec((1, D)), _full_spec((1, D))]
        + ([_full_spec((1, D)), _full_spec((1, D))] if inner else [])
        + [pl.BlockSpec(memory_space=pltpu.SMEM)]
    )
    return pl.pallas_call(
        body,
        grid=(NP // BLK,),
        in_specs=in_specs,
        out_specs=_row_spec(),
        out_shape=jax.ShapeDtypeStruct((NP, D), jnp.float32),
    )(h, a0, a1, w1, vecs[0], w2, vecs[1], vecs[2], vecs[3],
      *(vecs[4:] if inner else []), eps)


def kernel(x, edge_index, params):
    src = edge_index[0].astype(jnp.int32)
    dst = edge_index[1].astype(jnp.int32)
    pad = E_PAD - E
    if pad:
        src = jnp.concatenate([src, jnp.zeros((pad,), jnp.int32)])
        dst = jnp.concatenate([dst, jnp.full((pad,), N, jnp.int32)])
    ei = jnp.stack(
        [src.reshape(NW, CH, K), dst.reshape(NW, CH, K)], axis=2)
    zeros_rows = jnp.zeros((NP, D), jnp.float32)
    h = jnp.zeros((NP, D), jnp.float32).at[:N].set(x)
    for l in range(NLAYER):
        agg = _sc_agg()(h, ei, zeros_rows)
        inner = l < NLAYER - 1
        h = _tc_mlp(
            inner, h, agg[0], agg[1],
            params[f'W1_{l}'], params[f'b1_{l}'],
            params[f'W2_{l}'], params[f'b2_{l}'],
            params[f'ng_{l}'], params[f'nb_{l}'],
            params[f'sg_{l}'] if inner else None,
            params[f'sb_{l}'] if inner else None,
            params[f'eps_{l}'])
    return h[:N]


# speedup vs baseline: 2.9383x; 2.9383x over previous
import functools

import jax
import jax.numpy as jnp
from jax import lax
from jax.experimental import pallas as pl
from jax.experimental.pallas import tpu as pltpu
from jax.experimental.pallas import tpu_sc as plsc

N = 10000
E = 320000
D = 128
NLAYER = 3

NC = 2
NS = 16
NW = NC * NS

K = 128
GI = 8
NGI = 10
CH = GI * NGI
EPT = CH * K
E_PAD = EPT * NW

NP = 10112
RPT = NP // NS
BLK = 128


def _sc_agg_body(h_hbm, ei_hbm, zeros_hbm, out_hbm, ib, rows, acc,
                 isem0, isem1, gsem0, gsem1):
    isems = (isem0, isem1)
    gsems = (gsem0, gsem1)
    cid = lax.axis_index("c")
    sid = lax.axis_index("s")
    tid = sid * NC + cid

    pltpu.sync_copy(ei_hbm.at[tid, pl.ds(0, GI)], ib.at[0])
    pltpu.sync_copy(zeros_hbm.at[pl.ds(sid * RPT, RPT)],
                    acc.at[pl.ds(sid * RPT, RPT)])
    plsc.subcore_barrier()

    for b in range(2):
        pltpu.async_copy(h_hbm.at[ib.at[0, b, 0]], rows.at[b], gsems[b])

    def outer(go, carry):
        for half in range(2):
            g = go * 2 + half
            bi, bo = half, 1 - half

            @pl.when(g < NGI - 1)
            def _():
                pltpu.async_copy(ei_hbm.at[tid, pl.ds((g + 1) * GI, GI)],
                                 ib.at[bo], isems[bo])

            for j in range(GI):
                b = j % 2
                pltpu.make_async_copy(h_hbm.at[ib.at[bi, j, 0]],
                                      rows.at[b], gsems[b]).wait()
                pltpu.sync_copy(rows.at[b], acc.at[ib.at[bi, j, 1]],
                                add=True)
                if j == GI - 2:
                    @pl.when(g < NGI - 1)
                    def _():
                        pltpu.make_async_copy(
                            ei_hbm.at[tid, pl.ds((g + 1) * GI, GI)],
                            ib.at[bo], isems[bo]).wait()
                if j < GI - 2:
                    pltpu.async_copy(h_hbm.at[ib.at[bi, j + 2, 0]],
                                     rows.at[b], gsems[b])
                else:
                    @pl.when(g < NGI - 1)
                    def _():
                        pltpu.async_copy(
                            h_hbm.at[ib.at[bo, j + 2 - GI, 0]],
                            rows.at[b], gsems[b])
        return carry

    lax.fori_loop(0, NGI // 2, outer, 0)
    plsc.subcore_barrier()

    pltpu.sync_copy(acc.at[pl.ds(sid * RPT, RPT)],
                    out_hbm.at[cid, pl.ds(sid * RPT, RPT)])


@functools.cache
def _sc_agg():
    mesh = plsc.VectorSubcoreMesh(
        core_axis_name="c", subcore_axis_name="s",
        num_cores=NC, num_subcores=NS)
    return pl.kernel(
        _sc_agg_body,
        out_type=jax.ShapeDtypeStruct((NC, NP, D), jnp.float32),
        mesh=mesh,
        scratch_types=[
            pltpu.VMEM((2, GI, 2, K), jnp.int32),
            pltpu.VMEM((2, K, D), jnp.float32),
            pltpu.VMEM_SHARED((NP, D), jnp.float32),
            pltpu.SemaphoreType.DMA,
            pltpu.SemaphoreType.DMA,
            pltpu.SemaphoreType.DMA,
            pltpu.SemaphoreType.DMA,
        ],
    )


def _ln(x, g, b):
    mu = jnp.mean(x, axis=-1, keepdims=True)
    var = jnp.mean((x - mu) * (x - mu), axis=-1, keepdims=True)
    return (x - mu) * lax.rsqrt(var + 1e-5) * g + b


def _mlp_body_inner(h_ref, a0_ref, a1_ref, w1_ref, b1_ref, w2_ref, b2_ref,
                    ng_ref, nb_ref, sg_ref, sb_ref, eps_ref, out_ref):
    h = h_ref[...]
    z = (1.0 + eps_ref[0]) * h + a0_ref[...] + a1_ref[...]
    z = jnp.maximum(
        jnp.dot(z, w1_ref[...], preferred_element_type=jnp.float32)
        + b1_ref[...], 0.0)
    z = jnp.dot(z, w2_ref[...], preferred_element_type=jnp.float32) + b2_ref[...]
    z = _ln(z, ng_ref[...], nb_ref[...])
    z = jnp.maximum(_ln(z, sg_ref[...], sb_ref[...]), 0.0)
    out_ref[...] = z + h


def _mlp_body_last(h_ref, a0_ref, a1_ref, w1_ref, b1_ref, w2_ref, b2_ref,
                   ng_ref, nb_ref, eps_ref, out_ref):
    h = h_ref[...]
    z = (1.0 + eps_ref[0]) * h + a0_ref[...] + a1_ref[...]
    z = jnp.maximum(
        jnp.dot(z, w1_ref[...], preferred_element_type=jnp.float32)
        + b1_ref[...], 0.0)
    z = jnp.dot(z, w2_ref[...], preferred_element_type=jnp.float32) + b2_ref[...]
    z = _ln(z, ng_ref[...], nb_ref[...])
    out_ref[...] = z + h


def _row_spec():
    return pl.BlockSpec((BLK, D), lambda i: (i, 0))


def _full_spec(shape):
    nd = len(shape)
    return pl.BlockSpec(shape, lambda i: (0,) * nd)


def _tc_mlp(inner, h, a0, a1, w1, b1, w2, b2, ng, nb, sg, sb, eps):
    vecs = [v.reshape(1, D) for v in (b1, b2, ng, nb)]
    body = _mlp_body_last
    if inner:
        vecs += [sg.reshape(1, D), sb.reshape(1, D)]
        body = _mlp_body_inner
    in_specs = (
        [_row_spec(), _row_spec(), _row_spec(),
         _full_spec((D, D)), _full_spec((1, D)),
         _full_spec((D, D)), _full_spec((1, D)),
         _full_spec((1, D)), _full_spec((1, D))]
        + ([_full_spec((1, D)), _full_spec((1, D))] if inner else [])
        + [pl.BlockSpec(memory_space=pltpu.SMEM)]
    )
    return pl.pallas_call(
        body,
        grid=(NP // BLK,),
        in_specs=in_specs,
        out_specs=_row_spec(),
        out_shape=jax.ShapeDtypeStruct((NP, D), jnp.float32),
    )(h, a0, a1, w1, vecs[0], w2, vecs[1], vecs[2], vecs[3],
      *(vecs[4:] if inner else []), eps)


def kernel(x, edge_index, params):
    src = edge_index[0].astype(jnp.int32).reshape(NW, E // NW)
    dst = edge_index[1].astype(jnp.int32).reshape(NW, E // NW)
    ppt = EPT - E // NW
    pad_src = jnp.broadcast_to((jnp.arange(ppt) * 37) % N, (NW, ppt))
    pad_dst = N + (jnp.arange(ppt) % (NP - N))
    pad_dst = (pad_dst[None, :] + jnp.arange(NW)[:, None] * 7) % (NP - N) + N
    src = jnp.concatenate([src, pad_src.astype(jnp.int32)], axis=1)
    dst = jnp.concatenate([dst, pad_dst.astype(jnp.int32)], axis=1)
    ei = jnp.stack(
        [src.reshape(NW, CH, K), dst.reshape(NW, CH, K)], axis=2)
    zeros_rows = jnp.zeros((NP, D), jnp.float32)
    h = jnp.zeros((NP, D), jnp.float32).at[:N].set(x)
    for l in range(NLAYER):
        agg = _sc_agg()(h, ei, zeros_rows)
        inner = l < NLAYER - 1
        h = _tc_mlp(
            inner, h, agg[0], agg[1],
            params[f'W1_{l}'], params[f'b1_{l}'],
            params[f'W2_{l}'], params[f'b2_{l}'],
            params[f'ng_{l}'], params[f'nb_{l}'],
            params[f'sg_{l}'] if inner else None,
            params[f'sb_{l}'] if inner else None,
            params[f'eps_{l}'])
    return h[:N]


# speedup vs baseline: 3.0254x; 1.0296x over previous
import functools

import jax
import jax.numpy as jnp
from jax import lax
from jax.experimental import pallas as pl
from jax.experimental.pallas import tpu as pltpu
from jax.experimental.pallas import tpu_sc as plsc

N = 10000
E = 320000
D = 128
NLAYER = 3

NC = 2
NS = 16
NW = NC * NS

K = 120
NB = 3
GI = 6
NGI = 14
CH = GI * NGI
EPT = CH * K
E_PAD = EPT * NW

NP = 10112
RPT = NP // NS
BLK = 128


def _sc_agg_body(h_hbm, ei_hbm, zeros_hbm, out_hbm, ib, rows, acc,
                 isem0, isem1, gsem0, gsem1, gsem2, ssem0, ssem1, ssem2):
    isems = (isem0, isem1)
    gsems = (gsem0, gsem1, gsem2)
    ssems = (ssem0, ssem1, ssem2)
    cid = lax.axis_index("c")
    sid = lax.axis_index("s")
    tid = sid * NC + cid

    pltpu.sync_copy(ei_hbm.at[tid, pl.ds(0, GI)], ib.at[0])
    pltpu.sync_copy(zeros_hbm.at[pl.ds(sid * RPT, RPT)],
                    acc.at[pl.ds(sid * RPT, RPT)])
    plsc.subcore_barrier()

    for b in range(2):
        pltpu.async_copy(h_hbm.at[ib.at[0, b, 0]], rows.at[b], gsems[b])

    def outer(go, carry):
        for half in range(2):
            g = go * 2 + half
            bi, bo = half, 1 - half

            @pl.when(g < NGI - 1)
            def _():
                pltpu.async_copy(ei_hbm.at[tid, pl.ds((g + 1) * GI, GI)],
                                 ib.at[bo], isems[bo])

            for j in range(GI):
                b = j % NB
                pltpu.make_async_copy(h_hbm.at[ib.at[bi, j, 0]],
                                      rows.at[b], gsems[b]).wait()
                pltpu.async_copy(rows.at[b], acc.at[ib.at[bi, j, 1]],
                                 ssems[b], add=True)
                if j == GI - 2:
                    @pl.when(g < NGI - 1)
                    def _():
                        pltpu.make_async_copy(
                            ei_hbm.at[tid, pl.ds((g + 1) * GI, GI)],
                            ib.at[bo], isems[bo]).wait()
                b2 = (j + 2) % NB
                pb, pj = (bi, j - 1) if j > 0 else (bo, GI - 1)

                def wait_prev_scatter(pb=pb, pj=pj, b2=b2):
                    pltpu.make_async_copy(
                        rows.at[b2], acc.at[ib.at[pb, pj, 1]],
                        ssems[b2]).wait()

                if j == 0:
                    @pl.when(g > 0)
                    def _():
                        wait_prev_scatter()
                    pltpu.async_copy(h_hbm.at[ib.at[bi, j + 2, 0]],
                                     rows.at[b2], gsems[b2])
                elif j < GI - 2:
                    wait_prev_scatter()
                    pltpu.async_copy(h_hbm.at[ib.at[bi, j + 2, 0]],
                                     rows.at[b2], gsems[b2])
                else:
                    @pl.when(g < NGI - 1)
                    def _():
                        wait_prev_scatter()
                        pltpu.async_copy(
                            h_hbm.at[ib.at[bo, j + 2 - GI, 0]],
                            rows.at[b2], gsems[b2])
        return carry

    lax.fori_loop(0, NGI // 2, outer, 0)

    bi_last = (NGI - 1) % 2
    for jj in range(GI - NB, GI):
        pltpu.make_async_copy(rows.at[jj % NB],
                              acc.at[ib.at[bi_last, jj, 1]],
                              ssems[jj % NB]).wait()
    plsc.subcore_barrier()

    pltpu.sync_copy(acc.at[pl.ds(sid * RPT, RPT)],
                    out_hbm.at[cid, pl.ds(sid * RPT, RPT)])


@functools.cache
def _sc_agg():
    mesh = plsc.VectorSubcoreMesh(
        core_axis_name="c", subcore_axis_name="s",
        num_cores=NC, num_subcores=NS)
    return pl.kernel(
        _sc_agg_body,
        out_type=jax.ShapeDtypeStruct((NC, NP, D), jnp.float32),
        mesh=mesh,
        scratch_types=[
            pltpu.VMEM((2, GI, 2, K), jnp.int32),
            pltpu.VMEM((NB, K, D), jnp.float32),
            pltpu.VMEM_SHARED((NP, D), jnp.float32),
        ] + [pltpu.SemaphoreType.DMA] * (2 + 2 * NB),
    )


def _ln(x, g, b):
    mu = jnp.mean(x, axis=-1, keepdims=True)
    var = jnp.mean((x - mu) * (x - mu), axis=-1, keepdims=True)
    return (x - mu) * lax.rsqrt(var + 1e-5) * g + b


def _mlp_body_inner(h_ref, a0_ref, a1_ref, w1_ref, b1_ref, w2_ref, b2_ref,
                    ng_ref, nb_ref, sg_ref, sb_ref, eps_ref, out_ref):
    h = h_ref[...]
    z = (1.0 + eps_ref[0]) * h + a0_ref[...] + a1_ref[...]
    z = jnp.maximum(
        jnp.dot(z, w1_ref[...], preferred_element_type=jnp.float32)
        + b1_ref[...], 0.0)
    z = jnp.dot(z, w2_ref[...], preferred_element_type=jnp.float32) + b2_ref[...]
    z = _ln(z, ng_ref[...], nb_ref[...])
    z = jnp.maximum(_ln(z, sg_ref[...], sb_ref[...]), 0.0)
    out_ref[...] = z + h


def _mlp_body_last(h_ref, a0_ref, a1_ref, w1_ref, b1_ref, w2_ref, b2_ref,
                   ng_ref, nb_ref, eps_ref, out_ref):
    h = h_ref[...]
    z = (1.0 + eps_ref[0]) * h + a0_ref[...] + a1_ref[...]
    z = jnp.maximum(
        jnp.dot(z, w1_ref[...], preferred_element_type=jnp.float32)
        + b1_ref[...], 0.0)
    z = jnp.dot(z, w2_ref[...], preferred_element_type=jnp.float32) + b2_ref[...]
    z = _ln(z, ng_ref[...], nb_ref[...])
    out_ref[...] = z + h


def _row_spec():
    return pl.BlockSpec((BLK, D), lambda i: (i, 0))


def _full_spec(shape):
    nd = len(shape)
    return pl.BlockSpec(shape, lambda i: (0,) * nd)


def _tc_mlp(inner, h, a0, a1, w1, b1, w2, b2, ng, nb, sg, sb, eps):
    vecs = [v.reshape(1, D) for v in (b1, b2, ng, nb)]
    body = _mlp_body_last
    if inner:
        vecs += [sg.reshape(1, D), sb.reshape(1, D)]
        body = _mlp_body_inner
    in_specs = (
        [_row_spec(), _row_spec(), _row_spec(),
         _full_spec((D, D)), _full_spec((1, D)),
         _full_spec((D, D)), _full_spec((1, D)),
         _full_spec((1, D)), _full_spec((1, D))]
        + ([_full_spec((1, D)), _full_spec((1, D))] if inner else [])
        + [pl.BlockSpec(memory_space=pltpu.SMEM)]
    )
    return pl.pallas_call(
        body,
        grid=(NP // BLK,),
        in_specs=in_specs,
        out_specs=_row_spec(),
        out_shape=jax.ShapeDtypeStruct((NP, D), jnp.float32),
    )(h, a0, a1, w1, vecs[0], w2, vecs[1], vecs[2], vecs[3],
      *(vecs[4:] if inner else []), eps)


def kernel(x, edge_index, params):
    src = edge_index[0].astype(jnp.int32).reshape(NW, E // NW)
    dst = edge_index[1].astype(jnp.int32).reshape(NW, E // NW)
    ppt = EPT - E // NW
    pad_src = jnp.broadcast_to((jnp.arange(ppt) * 37) % N, (NW, ppt))
    pad_dst = N + (jnp.arange(ppt) % (NP - N))
    pad_dst = (pad_dst[None, :] + jnp.arange(NW)[:, None] * 7) % (NP - N) + N
    src = jnp.concatenate([src, pad_src.astype(jnp.int32)], axis=1)
    dst = jnp.concatenate([dst, pad_dst.astype(jnp.int32)], axis=1)
    ei = jnp.stack(
        [src.reshape(NW, CH, K), dst.reshape(NW, CH, K)], axis=2)
    zeros_rows = jnp.zeros((NP, D), jnp.float32)
    h = jnp.zeros((NP, D), jnp.float32).at[:N].set(x)
    for l in range(NLAYER):
        agg = _sc_agg()(h, ei, zeros_rows)
        inner = l < NLAYER - 1
        h = _tc_mlp(
            inner, h, agg[0], agg[1],
            params[f'W1_{l}'], params[f'b1_{l}'],
            params[f'W2_{l}'], params[f'b2_{l}'],
            params[f'ng_{l}'], params[f'nb_{l}'],
            params[f'sg_{l}'] if inner else None,
            params[f'sb_{l}'] if inner else None,
            params[f'eps_{l}'])
    return h[:N]


# speedup vs baseline: 4.0820x; 1.3492x over previous
import functools

import jax
import jax.numpy as jnp
from jax import lax
from jax.experimental import pallas as pl
from jax.experimental.pallas import tpu as pltpu
from jax.experimental.pallas import tpu_sc as plsc

N = 10000
E = 320000
D = 128
NLAYER = 3

NC = 2
NS = 16
NW = NC * NS

K = 120
NB = 3
GI = 6
NGI = 14
CH = GI * NGI
EPT = CH * K
E_PAD = EPT * NW

NP = 10112
RPT = NP // NS
BLK = 1264


def _sc_agg_body(h_hbm, ei_hbm, zeros_hbm, out_hbm, ib, rows, acc,
                 isem0, isem1, gsem0, gsem1, gsem2, ssem0, ssem1, ssem2):
    isems = (isem0, isem1)
    gsems = (gsem0, gsem1, gsem2)
    ssems = (ssem0, ssem1, ssem2)
    cid = lax.axis_index("c")
    sid = lax.axis_index("s")
    tid = sid * NC + cid

    pltpu.sync_copy(ei_hbm.at[tid, pl.ds(0, GI)], ib.at[0])
    pltpu.sync_copy(zeros_hbm.at[pl.ds(sid * RPT, RPT)],
                    acc.at[pl.ds(sid * RPT, RPT)])
    plsc.subcore_barrier()

    for b in range(2):
        pltpu.async_copy(h_hbm.at[ib.at[0, b, 0]], rows.at[b], gsems[b])

    def outer(go, carry):
        for half in range(2):
            g = go * 2 + half
            bi, bo = half, 1 - half

            @pl.when(g < NGI - 1)
            def _():
                pltpu.async_copy(ei_hbm.at[tid, pl.ds((g + 1) * GI, GI)],
                                 ib.at[bo], isems[bo])

            for j in range(GI):
                b = j % NB
                pltpu.make_async_copy(h_hbm.at[ib.at[bi, j, 0]],
                                      rows.at[b], gsems[b]).wait()
                pltpu.async_copy(rows.at[b], acc.at[ib.at[bi, j, 1]],
                                 ssems[b], add=True)
                if j == GI - 2:
                    @pl.when(g < NGI - 1)
                    def _():
                        pltpu.make_async_copy(
                            ei_hbm.at[tid, pl.ds((g + 1) * GI, GI)],
                            ib.at[bo], isems[bo]).wait()
                b2 = (j + 2) % NB
                pb, pj = (bi, j - 1) if j > 0 else (bo, GI - 1)

                def wait_prev_scatter(pb=pb, pj=pj, b2=b2):
                    pltpu.make_async_copy(
                        rows.at[b2], acc.at[ib.at[pb, pj, 1]],
                        ssems[b2]).wait()

                if j == 0:
                    @pl.when(g > 0)
                    def _():
                        wait_prev_scatter()
                    pltpu.async_copy(h_hbm.at[ib.at[bi, j + 2, 0]],
                                     rows.at[b2], gsems[b2])
                elif j < GI - 2:
                    wait_prev_scatter()
                    pltpu.async_copy(h_hbm.at[ib.at[bi, j + 2, 0]],
                                     rows.at[b2], gsems[b2])
                else:
                    @pl.when(g < NGI - 1)
                    def _():
                        wait_prev_scatter()
                        pltpu.async_copy(
                            h_hbm.at[ib.at[bo, j + 2 - GI, 0]],
                            rows.at[b2], gsems[b2])
        return carry

    lax.fori_loop(0, NGI // 2, outer, 0)

    bi_last = (NGI - 1) % 2
    for jj in range(GI - NB, GI):
        pltpu.make_async_copy(rows.at[jj % NB],
                              acc.at[ib.at[bi_last, jj, 1]],
                              ssems[jj % NB]).wait()
    plsc.subcore_barrier()

    pltpu.sync_copy(acc.at[pl.ds(sid * RPT, RPT)],
                    out_hbm.at[cid, pl.ds(sid * RPT, RPT)])


@functools.cache
def _sc_agg():
    mesh = plsc.VectorSubcoreMesh(
        core_axis_name="c", subcore_axis_name="s",
        num_cores=NC, num_subcores=NS)
    return pl.kernel(
        _sc_agg_body,
        out_type=jax.ShapeDtypeStruct((NC, NP, D), jnp.float32),
        mesh=mesh,
        scratch_types=[
            pltpu.VMEM((2, GI, 2, K), jnp.int32),
            pltpu.VMEM((NB, K, D), jnp.float32),
            pltpu.VMEM_SHARED((NP, D), jnp.float32),
        ] + [pltpu.SemaphoreType.DMA] * (2 + 2 * NB),
    )


def _ln(x, g, b):
    mu = jnp.mean(x, axis=-1, keepdims=True)
    var = jnp.mean((x - mu) * (x - mu), axis=-1, keepdims=True)
    return (x - mu) * lax.rsqrt(var + 1e-5) * g + b


def _mlp_body_inner(h_ref, a0_ref, a1_ref, w1_ref, b1_ref, w2_ref, b2_ref,
                    ng_ref, nb_ref, sg_ref, sb_ref, eps_ref, out_ref):
    h = h_ref[...]
    z = (1.0 + eps_ref[0]) * h + a0_ref[...] + a1_ref[...]
    z = jnp.maximum(
        jnp.dot(z, w1_ref[...], preferred_element_type=jnp.float32)
        + b1_ref[...], 0.0)
    z = jnp.dot(z, w2_ref[...], preferred_element_type=jnp.float32) + b2_ref[...]
    z = _ln(z, ng_ref[...], nb_ref[...])
    z = jnp.maximum(_ln(z, sg_ref[...], sb_ref[...]), 0.0)
    out_ref[...] = z + h


def _mlp_body_last(h_ref, a0_ref, a1_ref, w1_ref, b1_ref, w2_ref, b2_ref,
                   ng_ref, nb_ref, eps_ref, out_ref):
    h = h_ref[...]
    z = (1.0 + eps_ref[0]) * h + a0_ref[...] + a1_ref[...]
    z = jnp.maximum(
        jnp.dot(z, w1_ref[...], preferred_element_type=jnp.float32)
        + b1_ref[...], 0.0)
    z = jnp.dot(z, w2_ref[...], preferred_element_type=jnp.float32) + b2_ref[...]
    z = _ln(z, ng_ref[...], nb_ref[...])
    out_ref[...] = z + h


def _row_spec():
    return pl.BlockSpec((BLK, D), lambda i: (i, 0))


def _full_spec(shape):
    nd = len(shape)
    return pl.BlockSpec(shape, lambda i: (0,) * nd)


def _tc_mlp(inner, h, a0, a1, w1, b1, w2, b2, ng, nb, sg, sb, eps):
    vecs = [v.reshape(1, D) for v in (b1, b2, ng, nb)]
    body = _mlp_body_last
    if inner:
        vecs += [sg.reshape(1, D), sb.reshape(1, D)]
        body = _mlp_body_inner
    in_specs = (
        [_row_spec(), _row_spec(), _row_spec(),
         _full_spec((D, D)), _full_spec((1, D)),
         _full_spec((D, D)), _full_spec((1, D)),
         _full_spec((1, D)), _full_spec((1, D))]
        + ([_full_spec((1, D)), _full_spec((1, D))] if inner else [])
        + [pl.BlockSpec(memory_space=pltpu.SMEM)]
    )
    return pl.pallas_call(
        body,
        grid=(NP // BLK,),
        in_specs=in_specs,
        out_specs=_row_spec(),
        out_shape=jax.ShapeDtypeStruct((NP, D), jnp.float32),
    )(h, a0, a1, w1, vecs[0], w2, vecs[1], vecs[2], vecs[3],
      *(vecs[4:] if inner else []), eps)


def kernel(x, edge_index, params):
    src = edge_index[0].astype(jnp.int32).reshape(NW, E // NW)
    dst = edge_index[1].astype(jnp.int32).reshape(NW, E // NW)
    ppt = EPT - E // NW
    pad_src = jnp.broadcast_to((jnp.arange(ppt) * 37) % N, (NW, ppt))
    pad_dst = N + (jnp.arange(ppt) % (NP - N))
    pad_dst = (pad_dst[None, :] + jnp.arange(NW)[:, None] * 7) % (NP - N) + N
    src = jnp.concatenate([src, pad_src.astype(jnp.int32)], axis=1)
    dst = jnp.concatenate([dst, pad_dst.astype(jnp.int32)], axis=1)
    ei = jnp.stack(
        [src.reshape(NW, CH, K), dst.reshape(NW, CH, K)], axis=2)
    zeros_rows = jnp.zeros((NP, D), jnp.float32)
    h = jnp.zeros((NP, D), jnp.float32).at[:N].set(x)
    for l in range(NLAYER):
        agg = _sc_agg()(h, ei, zeros_rows)
        inner = l < NLAYER - 1
        h = _tc_mlp(
            inner, h, agg[0], agg[1],
            params[f'W1_{l}'], params[f'b1_{l}'],
            params[f'W2_{l}'], params[f'b2_{l}'],
            params[f'ng_{l}'], params[f'nb_{l}'],
            params[f'sg_{l}'] if inner else None,
            params[f'sb_{l}'] if inner else None,
            params[f'eps_{l}'])
    return h[:N]


# speedup vs baseline: 4.4056x; 1.0793x over previous
import functools

import jax
import jax.numpy as jnp
from jax import lax
from jax.experimental import pallas as pl
from jax.experimental.pallas import tpu as pltpu
from jax.experimental.pallas import tpu_sc as plsc

N = 10000
E = 320000
D = 128
NLAYER = 3

NC = 2
NS = 16
NW = NC * NS

K = 125
GI = 8
NGI = 10
CH = GI * NGI
EPT = CH * K

NP = 10112
RPT = NP // NS
BLK = 1264


def _sc_agg_body(h_hbm, ei_hbm, zeros_hbm, out_hbm, ib, rows, acc,
                 isem0, isem1, gsem0, gsem1):
    isems = (isem0, isem1)
    gsems = (gsem0, gsem1)
    cid = lax.axis_index("c")
    sid = lax.axis_index("s")
    tid = sid * NC + cid

    pltpu.sync_copy(ei_hbm.at[0, tid, pl.ds(0, GI)], ib.at[0, 0])
    pltpu.sync_copy(ei_hbm.at[1, tid, pl.ds(0, GI)], ib.at[0, 1])
    pltpu.sync_copy(zeros_hbm.at[pl.ds(sid * RPT, RPT)],
                    acc.at[pl.ds(sid * RPT, RPT)])
    plsc.subcore_barrier()

    for b in range(2):
        pltpu.async_copy(h_hbm.at[ib.at[0, 0, b]], rows.at[b], gsems[b])

    def outer(go, carry):
        for half in range(2):
            g = go * 2 + half
            bi, bo = half, 1 - half

            @pl.when(g < NGI - 1)
            def _():
                pltpu.async_copy(ei_hbm.at[0, tid, pl.ds((g + 1) * GI, GI)],
                                 ib.at[bo, 0], isems[bo])
                pltpu.async_copy(ei_hbm.at[1, tid, pl.ds((g + 1) * GI, GI)],
                                 ib.at[bo, 1], isems[bo])

            for j in range(GI):
                b = j % 2
                pltpu.make_async_copy(h_hbm.at[ib.at[bi, 0, j]],
                                      rows.at[b], gsems[b]).wait()
                pltpu.sync_copy(rows.at[b], acc.at[ib.at[bi, 1, j]],
                                add=True)
                if j == GI - 2:
                    @pl.when(g < NGI - 1)
                    def _():
                        pltpu.make_async_copy(
                            ei_hbm.at[0, tid, pl.ds((g + 1) * GI, GI)],
                            ib.at[bo, 0], isems[bo]).wait()
                        pltpu.make_async_copy(
                            ei_hbm.at[1, tid, pl.ds((g + 1) * GI, GI)],
                            ib.at[bo, 1], isems[bo]).wait()
                if j < GI - 2:
                    pltpu.async_copy(h_hbm.at[ib.at[bi, 0, j + 2]],
                                     rows.at[b], gsems[b])
                else:
                    @pl.when(g < NGI - 1)
                    def _():
                        pltpu.async_copy(
                            h_hbm.at[ib.at[bo, 0, j + 2 - GI]],
                            rows.at[b], gsems[b])
        return carry

    lax.fori_loop(0, NGI // 2, outer, 0)
    plsc.subcore_barrier()

    pltpu.sync_copy(acc.at[pl.ds(sid * RPT, RPT)],
                    out_hbm.at[cid, pl.ds(sid * RPT, RPT)])


@functools.cache
def _sc_agg():
    mesh = plsc.VectorSubcoreMesh(
        core_axis_name="c", subcore_axis_name="s",
        num_cores=NC, num_subcores=NS)
    return pl.kernel(
        _sc_agg_body,
        out_type=jax.ShapeDtypeStruct((NC, NP, D), jnp.float32),
        mesh=mesh,
        scratch_types=[
            pltpu.VMEM((2, 2, GI, K), jnp.int32),
            pltpu.VMEM((2, K, D), jnp.float32),
            pltpu.VMEM_SHARED((NP, D), jnp.float32),
        ] + [pltpu.SemaphoreType.DMA] * 4,
    )


def _ln(x, g, b):
    mu = jnp.mean(x, axis=-1, keepdims=True)
    var = jnp.mean((x - mu) * (x - mu), axis=-1, keepdims=True)
    return (x - mu) * lax.rsqrt(var + 1e-5) * g + b


def _mlp_body_inner(h_ref, a_ref, w1_ref, b1_ref, w2_ref, b2_ref,
                    ng_ref, nb_ref, sg_ref, sb_ref, eps_ref, out_ref):
    h = h_ref[...]
    z = (1.0 + eps_ref[0]) * h + a_ref[0] + a_ref[1]
    z = jnp.maximum(
        jnp.dot(z, w1_ref[...], preferred_element_type=jnp.float32)
        + b1_ref[...], 0.0)
    z = jnp.dot(z, w2_ref[...], preferred_element_type=jnp.float32) + b2_ref[...]
    z = _ln(z, ng_ref[...], nb_ref[...])
    z = jnp.maximum(_ln(z, sg_ref[...], sb_ref[...]), 0.0)
    out_ref[...] = z + h


def _mlp_body_last(h_ref, a_ref, w1_ref, b1_ref, w2_ref, b2_ref,
                   ng_ref, nb_ref, eps_ref, out_ref):
    h = h_ref[...]
    z = (1.0 + eps_ref[0]) * h + a_ref[0] + a_ref[1]
    z = jnp.maximum(
        jnp.dot(z, w1_ref[...], preferred_element_type=jnp.float32)
        + b1_ref[...], 0.0)
    z = jnp.dot(z, w2_ref[...], preferred_element_type=jnp.float32) + b2_ref[...]
    z = _ln(z, ng_ref[...], nb_ref[...])
    out_ref[...] = z + h


def _row_spec():
    return pl.BlockSpec((BLK, D), lambda i: (i, 0))


def _full_spec(shape):
    nd = len(shape)
    return pl.BlockSpec(shape, lambda i: (0,) * nd)


def _tc_mlp(inner, h, agg, w1, b1, w2, b2, ng, nb, sg, sb, eps):
    vecs = [v.reshape(1, D) for v in (b1, b2, ng, nb)]
    body = _mlp_body_last
    if inner:
        vecs += [sg.reshape(1, D), sb.reshape(1, D)]
        body = _mlp_body_inner
    in_specs = (
        [_row_spec(),
         pl.BlockSpec((2, BLK, D), lambda i: (0, i, 0)),
         _full_spec((D, D)), _full_spec((1, D)),
         _full_spec((D, D)), _full_spec((1, D)),
         _full_spec((1, D)), _full_spec((1, D))]
        + ([_full_spec((1, D)), _full_spec((1, D))] if inner else [])
        + [pl.BlockSpec(memory_space=pltpu.SMEM)]
    )
    n_out = NP if inner else N
    return pl.pallas_call(
        body,
        grid=(NP // BLK,),
        in_specs=in_specs,
        out_specs=_row_spec(),
        out_shape=jax.ShapeDtypeStruct((n_out, D), jnp.float32),
    )(h, agg, w1, vecs[0], w2, vecs[1], vecs[2], vecs[3],
      *(vecs[4:] if inner else []), eps)


def kernel(x, edge_index, params):
    ei = edge_index.astype(jnp.int32).reshape(2, NW, CH, K)
    zeros_rows = jnp.zeros((NP, D), jnp.float32)
    h = jnp.zeros((NP, D), jnp.float32).at[:N].set(x)
    for l in range(NLAYER):
        agg = _sc_agg()(h, ei, zeros_rows)
        inner = l < NLAYER - 1
        h = _tc_mlp(
            inner, h, agg,
            params[f'W1_{l}'], params[f'b1_{l}'],
            params[f'W2_{l}'], params[f'b2_{l}'],
            params[f'ng_{l}'], params[f'nb_{l}'],
            params[f'sg_{l}'] if inner else None,
            params[f'sb_{l}'] if inner else None,
            params[f'eps_{l}'])
    return h
